# baseline (speedup 1.0000x reference)
import functools

import jax
import jax.numpy as jnp
from jax import lax
from jax.experimental import pallas as pl
from jax.experimental.pallas import tpu as pltpu
from jax.experimental.pallas import tpu_sc as plsc

N_NODES = 10000
N_EDGES = 320000
IN_DIM = 128
HIDDEN_DIM = 128
EPS = 1e-5

NC = 2
NS = 16
HF = IN_DIM // NC
C = 128
K = 157
E_PAD = NS * K * C
NB = 10
NI = NB + 1
ACC_ROWS = N_NODES + 112
ZROWS = ACC_ROWS // NS


def _sc_aggregate(h2, eidx, zeros):
    mesh = plsc.VectorSubcoreMesh(core_axis_name="c", subcore_axis_name="s")

    @functools.partial(
        pl.kernel,
        out_type=jax.ShapeDtypeStruct((NC, ACC_ROWS, HF), jnp.float32),
        mesh=mesh,
        scratch_types=[
            pltpu.VMEM((NI, 3, C), jnp.int32),
            pltpu.VMEM((NB, C, HF), jnp.float32),
            pltpu.VMEM_SHARED((ACC_ROWS, HF), jnp.float32),
            pltpu.SemaphoreType.DMA,
            pltpu.SemaphoreType.DMA,
            pltpu.SemaphoreType.DMA,
        ],
        compiler_params=pltpu.CompilerParams(use_tc_tiling_on_sc=False),
    )
    def sc_kernel(h_hbm, eidx_hbm, z_hbm, out_hbm, idx_v, rows_v, acc_sh,
                  sem_g, sem_i, sem_s):
        c = lax.axis_index("c")
        s = lax.axis_index("s")
        pltpu.sync_copy(z_hbm, acc_sh.at[pl.ds(s * ZROWS, ZROWS)])
        plsc.subcore_barrier()

        for i in range(NB - 1):
            pltpu.sync_copy(eidx_hbm.at[s, i], idx_v.at[i])
            pltpu.async_copy(h_hbm.at[idx_v.at[i, c]], rows_v.at[i],
                             sem_g)
        pltpu.async_copy(eidx_hbm.at[s, NB - 1], idx_v.at[NB - 1], sem_i)

        def body(j, carry):
            cur = lax.rem(j, NB)
            curi = lax.rem(j, NI)
            prv = lax.rem(j + NB - 1, NB)
            prvi = lax.rem(j + NB, NI)
            nxgi = lax.rem(j + NB - 1, NI)

            @pl.when(j >= 1)
            def _():
                pltpu.make_async_copy(rows_v.at[prv],
                                      acc_sh.at[idx_v.at[prvi, 2]],
                                      sem_s).wait()

            @pl.when(j + NB - 1 < K)
            def _():
                pltpu.make_async_copy(eidx_hbm.at[s, j + NB - 1],
                                      idx_v.at[nxgi], sem_i).wait()
                pltpu.async_copy(h_hbm.at[idx_v.at[nxgi, c]],
                                 rows_v.at[prv], sem_g)

            @pl.when(j + NB < K)
            def _():
                pltpu.async_copy(eidx_hbm.at[s, j + NB], idx_v.at[prvi],
                                 sem_i)

            pltpu.make_async_copy(h_hbm.at[idx_v.at[curi, c]],
                                  rows_v.at[cur], sem_g).wait()
            pltpu.async_copy(rows_v.at[cur], acc_sh.at[idx_v.at[curi, 2]],
                             sem_s, add=True)
            return carry

        lax.fori_loop(0, K, body, 0, unroll=2)
        pltpu.make_async_copy(rows_v.at[(K - 1) % NB],
                              acc_sh.at[idx_v.at[(K - 1) % NI, 2]],
                              sem_s).wait()
        plsc.subcore_barrier()
        pltpu.sync_copy(acc_sh.at[pl.ds(s * ZROWS, ZROWS)],
                        out_hbm.at[c, pl.ds(s * ZROWS, ZROWS)])

    return sc_kernel(h2, eidx, zeros)


def _tc_finish(partials, W, b, gamma, beta):
    def body(p_ref, w_ref, b_ref, g_ref, be_ref, o_ref):
        agg = jnp.concatenate(
            [p_ref[0, :N_NODES, :], p_ref[1, :N_NODES, :]], axis=1)
        out = lax.dot_general(agg, w_ref[...], (((1,), (1,)), ((), ())),
                              preferred_element_type=jnp.float32)
        out = out + b_ref[...]
        mean = jnp.mean(out, axis=0, keepdims=True)
        var = jnp.mean((out - mean) ** 2, axis=0, keepdims=True)
        o_ref[...] = (out - mean) * lax.rsqrt(var + EPS) * g_ref[...] + be_ref[...]

    return pl.pallas_call(
        body,
        out_shape=jax.ShapeDtypeStruct((N_NODES, HIDDEN_DIM), jnp.float32),
    )(partials, W, b.reshape(1, HIDDEN_DIM), gamma.reshape(1, HIDDEN_DIM),
      beta.reshape(1, HIDDEN_DIM))


def kernel(h, edge_index, W, b, gamma, beta):
    src = edge_index[0].astype(jnp.int32)
    dst = edge_index[1].astype(jnp.int32)
    pad = E_PAD - N_EDGES
    src = jnp.concatenate([src, jnp.zeros((pad,), jnp.int32)])
    dst = jnp.concatenate([dst, jnp.full((pad,), N_NODES, jnp.int32)])
    src2 = (src * 2).reshape(NS, K, C)
    eidx = jnp.stack([src2, src2 + 1, dst.reshape(NS, K, C)], axis=2)
    h2 = h.reshape(2 * N_NODES, HF)
    zeros = jnp.zeros((ZROWS, HF), jnp.float32)
    partials = _sc_aggregate(h2, eidx, zeros)
    return _tc_finish(partials, W, b, gamma, beta)


# speedup vs baseline: 1.0447x; 1.0447x over previous
import functools

import jax
import jax.numpy as jnp
from jax import lax
from jax.experimental import pallas as pl
from jax.experimental.pallas import tpu as pltpu
from jax.experimental.pallas import tpu_sc as plsc

N_NODES = 10000
N_EDGES = 320000
IN_DIM = 128
HIDDEN_DIM = 128
EPS = 1e-5

NC = 2
NS = 16
HF = IN_DIM // NC
C = 128
K = 157
E_PAD = NS * K * C
NB = 10
NI = NB + 1
ACC_ROWS = N_NODES + 112
ZROWS = ACC_ROWS // NS


def _sc_aggregate(h_split, eidx, zeros):
    mesh = plsc.VectorSubcoreMesh(core_axis_name="c", subcore_axis_name="s")

    @functools.partial(
        pl.kernel,
        out_type=jax.ShapeDtypeStruct((NC, ACC_ROWS, HF), jnp.float32),
        mesh=mesh,
        scratch_types=[
            pltpu.VMEM((NI, 2, C), jnp.int32),
            pltpu.VMEM((NB, C, HF), jnp.float32),
            pltpu.VMEM_SHARED((ACC_ROWS, HF), jnp.float32),
            pltpu.SemaphoreType.DMA,
            pltpu.SemaphoreType.DMA,
            pltpu.SemaphoreType.DMA,
        ],
        compiler_params=pltpu.CompilerParams(use_tc_tiling_on_sc=False),
    )
    def sc_kernel(h_hbm, eidx_hbm, z_hbm, out_hbm, idx_v, rows_v, acc_sh,
                  sem_g, sem_i, sem_s):
        c = lax.axis_index("c")
        s = lax.axis_index("s")
        pltpu.sync_copy(z_hbm.at[pl.ds(s * ZROWS, ZROWS)],
                        acc_sh.at[pl.ds(s * ZROWS, ZROWS)])
        plsc.subcore_barrier()

        for i in range(NB - 1):
            pltpu.sync_copy(eidx_hbm.at[s, i], idx_v.at[i])
            pltpu.async_copy(h_hbm.at[c].at[idx_v.at[i, 0]], rows_v.at[i],
                             sem_g)
        pltpu.async_copy(eidx_hbm.at[s, NB - 1], idx_v.at[NB - 1], sem_i)

        def body(j, carry):
            cur = lax.rem(j, NB)
            curi = lax.rem(j, NI)
            prv = lax.rem(j + NB - 1, NB)
            prvi = lax.rem(j + NB, NI)
            nxgi = lax.rem(j + NB - 1, NI)

            @pl.when(j >= 1)
            def _():
                pltpu.make_async_copy(rows_v.at[prv],
                                      acc_sh.at[idx_v.at[prvi, 1]],
                                      sem_s).wait()

            @pl.when(j + NB - 1 < K)
            def _():
                pltpu.make_async_copy(eidx_hbm.at[s, j + NB - 1],
                                      idx_v.at[nxgi], sem_i).wait()
                pltpu.async_copy(h_hbm.at[c].at[idx_v.at[nxgi, 0]],
                                 rows_v.at[prv], sem_g)

            @pl.when(j + NB < K)
            def _():
                pltpu.async_copy(eidx_hbm.at[s, j + NB], idx_v.at[prvi],
                                 sem_i)

            pltpu.make_async_copy(h_hbm.at[c].at[idx_v.at[curi, 0]],
                                  rows_v.at[cur], sem_g).wait()
            pltpu.async_copy(rows_v.at[cur], acc_sh.at[idx_v.at[curi, 1]],
                             sem_s, add=True)
            return carry

        lax.fori_loop(0, K, body, 0, unroll=2)
        pltpu.make_async_copy(rows_v.at[(K - 1) % NB],
                              acc_sh.at[idx_v.at[(K - 1) % NI, 1]],
                              sem_s).wait()
        plsc.subcore_barrier()
        pltpu.sync_copy(acc_sh.at[pl.ds(s * ZROWS, ZROWS)],
                        out_hbm.at[c, pl.ds(s * ZROWS, ZROWS)])

    return sc_kernel(h_split, eidx, zeros)


def _split_h(h):
    BR = 1000

    def body(h_ref, o_ref):
        o_ref[0] = h_ref[:, :HF]
        o_ref[1] = h_ref[:, HF:]

    return pl.pallas_call(
        body,
        grid=(N_NODES // BR,),
        in_specs=[pl.BlockSpec((BR, IN_DIM), lambda i: (i, 0))],
        out_specs=pl.BlockSpec((NC, BR, HF), lambda i: (0, i, 0)),
        out_shape=jax.ShapeDtypeStruct((NC, N_NODES, HF), jnp.float32),
    )(h)


def _tc_finish(partials, W, b, gamma, beta):
    def body(p_ref, w_ref, b_ref, g_ref, be_ref, o_ref):
        agg = jnp.concatenate(
            [p_ref[0, :N_NODES, :], p_ref[1, :N_NODES, :]], axis=1)
        out = lax.dot_general(agg, w_ref[...], (((1,), (1,)), ((), ())),
                              preferred_element_type=jnp.float32)
        out = out + b_ref[...]
        mean = jnp.mean(out, axis=0, keepdims=True)
        var = jnp.mean((out - mean) ** 2, axis=0, keepdims=True)
        o_ref[...] = (out - mean) * lax.rsqrt(var + EPS) * g_ref[...] + be_ref[...]

    return pl.pallas_call(
        body,
        out_shape=jax.ShapeDtypeStruct((N_NODES, HIDDEN_DIM), jnp.float32),
    )(partials, W, b.reshape(1, HIDDEN_DIM), gamma.reshape(1, HIDDEN_DIM),
      beta.reshape(1, HIDDEN_DIM))


def kernel(h, edge_index, W, b, gamma, beta):
    src = edge_index[0].astype(jnp.int32)
    dst = edge_index[1].astype(jnp.int32)
    pad = E_PAD - N_EDGES
    src = jnp.concatenate([src, jnp.zeros((pad,), jnp.int32)])
    dst = jnp.concatenate([dst, jnp.full((pad,), N_NODES, jnp.int32)])
    eidx = jnp.stack([src.reshape(NS, K, C), dst.reshape(NS, K, C)], axis=2)
    h_split = _split_h(h)
    zeros = jnp.zeros((ACC_ROWS, HF), jnp.float32)
    partials = _sc_aggregate(h_split, eidx, zeros)
    return _tc_finish(partials, W, b, gamma, beta)


# speedup vs baseline: 1.0618x; 1.0164x over previous
import functools

import jax
import jax.numpy as jnp
from jax import lax
from jax.experimental import pallas as pl
from jax.experimental.pallas import tpu as pltpu
from jax.experimental.pallas import tpu_sc as plsc

N_NODES = 10000
N_EDGES = 320000
IN_DIM = 128
HIDDEN_DIM = 128
EPS = 1e-5

NC = 2
NS = 16
HF = IN_DIM // NC
C = 128
K = 157
E_PAD = NS * K * C
NB = 10
NI = NB + 1
ACC_ROWS = N_NODES + 112
ZROWS = ACC_ROWS // NS


def _sc_aggregate(h_split, eidx, zeros):
    mesh = plsc.VectorSubcoreMesh(core_axis_name="c", subcore_axis_name="s")

    @functools.partial(
        pl.kernel,
        out_type=jax.ShapeDtypeStruct((NC, ACC_ROWS, HF), jnp.float32),
        mesh=mesh,
        scratch_types=[
            pltpu.VMEM((NI, 2, C), jnp.int32),
            pltpu.VMEM((NB, C, HF), jnp.float32),
            pltpu.VMEM_SHARED((ACC_ROWS, HF), jnp.float32),
            pltpu.SemaphoreType.DMA,
            pltpu.SemaphoreType.DMA,
            pltpu.SemaphoreType.DMA,
        ],
        compiler_params=pltpu.CompilerParams(use_tc_tiling_on_sc=False),
    )
    def sc_kernel(h_hbm, eidx_hbm, z_hbm, out_hbm, idx_v, rows_v, acc_sh,
                  sem_g, sem_i, sem_s):
        c = lax.axis_index("c")
        s = lax.axis_index("s")
        pltpu.sync_copy(z_hbm.at[pl.ds(s * ZROWS, ZROWS)],
                        acc_sh.at[pl.ds(s * ZROWS, ZROWS)])
        plsc.subcore_barrier()

        for i in range(NB - 1):
            pltpu.sync_copy(eidx_hbm.at[s, i], idx_v.at[i])
            pltpu.async_copy(h_hbm.at[c].at[idx_v.at[i, 0]], rows_v.at[i],
                             sem_g)
        pltpu.async_copy(eidx_hbm.at[s, NB - 1], idx_v.at[NB - 1], sem_i)

        def body(j, carry):
            cur = lax.rem(j, NB)
            curi = lax.rem(j, NI)
            prv = lax.rem(j + NB - 1, NB)
            prvi = lax.rem(j + NB, NI)
            nxgi = lax.rem(j + NB - 1, NI)

            @pl.when(j >= 1)
            def _():
                pltpu.make_async_copy(rows_v.at[prv],
                                      acc_sh.at[idx_v.at[prvi, 1]],
                                      sem_s).wait()

            @pl.when(j + NB - 1 < K)
            def _():
                pltpu.make_async_copy(eidx_hbm.at[s, j + NB - 1],
                                      idx_v.at[nxgi], sem_i).wait()
                pltpu.async_copy(h_hbm.at[c].at[idx_v.at[nxgi, 0]],
                                 rows_v.at[prv], sem_g)

            @pl.when(j + NB < K)
            def _():
                pltpu.async_copy(eidx_hbm.at[s, j + NB], idx_v.at[prvi],
                                 sem_i)

            pltpu.make_async_copy(h_hbm.at[c].at[idx_v.at[curi, 0]],
                                  rows_v.at[cur], sem_g).wait()
            pltpu.async_copy(rows_v.at[cur], acc_sh.at[idx_v.at[curi, 1]],
                             sem_s, add=True)
            return carry

        lax.fori_loop(0, K, body, 0, unroll=2)
        pltpu.make_async_copy(rows_v.at[(K - 1) % NB],
                              acc_sh.at[idx_v.at[(K - 1) % NI, 1]],
                              sem_s).wait()
        plsc.subcore_barrier()
        pltpu.sync_copy(acc_sh.at[pl.ds(s * ZROWS, ZROWS)],
                        out_hbm.at[c, pl.ds(s * ZROWS, ZROWS)])

    return sc_kernel(h_split, eidx, zeros)


def _tc_finish(partials, W, b, gamma, beta):
    def body(p_ref, w_ref, b_ref, g_ref, be_ref, o_ref):
        agg = jnp.concatenate(
            [p_ref[0, :N_NODES, :], p_ref[1, :N_NODES, :]], axis=1)
        out = lax.dot_general(agg, w_ref[...], (((1,), (1,)), ((), ())),
                              preferred_element_type=jnp.float32)
        out = out + b_ref[...]
        mean = jnp.mean(out, axis=0, keepdims=True)
        var = jnp.mean((out - mean) ** 2, axis=0, keepdims=True)
        o_ref[...] = (out - mean) * lax.rsqrt(var + EPS) * g_ref[...] + be_ref[...]

    return pl.pallas_call(
        body,
        out_shape=jax.ShapeDtypeStruct((N_NODES, HIDDEN_DIM), jnp.float32),
    )(partials, W, b.reshape(1, HIDDEN_DIM), gamma.reshape(1, HIDDEN_DIM),
      beta.reshape(1, HIDDEN_DIM))


def kernel(h, edge_index, W, b, gamma, beta):
    src = edge_index[0].astype(jnp.int32)
    dst = edge_index[1].astype(jnp.int32)
    pad = E_PAD - N_EDGES
    src = jnp.concatenate([src, jnp.zeros((pad,), jnp.int32)])
    dst = jnp.concatenate([dst, jnp.full((pad,), N_NODES, jnp.int32)])
    eidx = jnp.stack([src.reshape(NS, K, C), dst.reshape(NS, K, C)], axis=2)
    h_split = jnp.stack([h[:, :HF], h[:, HF:]])
    zeros = jnp.zeros((ACC_ROWS, HF), jnp.float32)
    partials = _sc_aggregate(h_split, eidx, zeros)
    return _tc_finish(partials, W, b, gamma, beta)


# speedup vs baseline: 1.0680x; 1.0059x over previous
import functools

import jax
import jax.numpy as jnp
from jax import lax
from jax.experimental import pallas as pl
from jax.experimental.pallas import tpu as pltpu
from jax.experimental.pallas import tpu_sc as plsc

N_NODES = 10000
N_EDGES = 320000
IN_DIM = 128
HIDDEN_DIM = 128
EPS = 1e-5

NC = 2
NS = 16
HF = IN_DIM // NC
C = 128
K = 157
E_PAD = NS * K * C
NB = 10
NI = NB + 1
ACC_ROWS = N_NODES + 112
ZROWS = ACC_ROWS // NS


def _sc_aggregate(h_split, eidx, zeros):
    mesh = plsc.VectorSubcoreMesh(core_axis_name="c", subcore_axis_name="s")

    @functools.partial(
        pl.kernel,
        out_type=jax.ShapeDtypeStruct((NC, ACC_ROWS, HF), jnp.float32),
        mesh=mesh,
        scratch_types=[
            pltpu.VMEM((NI, 2, C), jnp.int32),
            pltpu.VMEM((NB, C, HF), jnp.float32),
            pltpu.VMEM_SHARED((ACC_ROWS, HF), jnp.float32),
            pltpu.SemaphoreType.DMA,
            pltpu.SemaphoreType.DMA,
            pltpu.SemaphoreType.DMA,
        ],
        compiler_params=pltpu.CompilerParams(use_tc_tiling_on_sc=False),
    )
    def sc_kernel(h_hbm, eidx_hbm, z_hbm, out_hbm, idx_v, rows_v, acc_sh,
                  sem_g, sem_i, sem_s):
        c = lax.axis_index("c")
        s = lax.axis_index("s")
        pltpu.sync_copy(eidx_hbm.at[s, pl.ds(0, NB - 1)],
                        idx_v.at[pl.ds(0, NB - 1)])
        for i in range(NB - 1):
            pltpu.async_copy(h_hbm.at[c].at[idx_v.at[i, 0]], rows_v.at[i],
                             sem_g)
        pltpu.async_copy(eidx_hbm.at[s, NB - 1], idx_v.at[NB - 1], sem_i)
        pltpu.sync_copy(z_hbm.at[pl.ds(s * ZROWS, ZROWS)],
                        acc_sh.at[pl.ds(s * ZROWS, ZROWS)])
        plsc.subcore_barrier()

        def body(j, carry):
            cur = lax.rem(j, NB)
            curi = lax.rem(j, NI)
            prv = lax.rem(j + NB - 1, NB)
            prvi = lax.rem(j + NB, NI)
            nxgi = lax.rem(j + NB - 1, NI)

            @pl.when(j >= 1)
            def _():
                pltpu.make_async_copy(rows_v.at[prv],
                                      acc_sh.at[idx_v.at[prvi, 1]],
                                      sem_s).wait()

            @pl.when(j + NB - 1 < K)
            def _():
                pltpu.make_async_copy(eidx_hbm.at[s, j + NB - 1],
                                      idx_v.at[nxgi], sem_i).wait()
                pltpu.async_copy(h_hbm.at[c].at[idx_v.at[nxgi, 0]],
                                 rows_v.at[prv], sem_g)

            @pl.when(j + NB < K)
            def _():
                pltpu.async_copy(eidx_hbm.at[s, j + NB], idx_v.at[prvi],
                                 sem_i)

            pltpu.make_async_copy(h_hbm.at[c].at[idx_v.at[curi, 0]],
                                  rows_v.at[cur], sem_g).wait()
            pltpu.async_copy(rows_v.at[cur], acc_sh.at[idx_v.at[curi, 1]],
                             sem_s, add=True)
            return carry

        lax.fori_loop(0, K, body, 0, unroll=2)
        pltpu.make_async_copy(rows_v.at[(K - 1) % NB],
                              acc_sh.at[idx_v.at[(K - 1) % NI, 1]],
                              sem_s).wait()
        plsc.subcore_barrier()
        pltpu.sync_copy(acc_sh.at[pl.ds(s * ZROWS, ZROWS)],
                        out_hbm.at[c, pl.ds(s * ZROWS, ZROWS)])

    return sc_kernel(h_split, eidx, zeros)


def _tc_finish(partials, W, b, gamma, beta):
    def body(p_ref, w_ref, b_ref, g_ref, be_ref, o_ref):
        agg = jnp.concatenate(
            [p_ref[0, :N_NODES, :], p_ref[1, :N_NODES, :]], axis=1)
        out = lax.dot_general(agg, w_ref[...], (((1,), (1,)), ((), ())),
                              preferred_element_type=jnp.float32)
        out = out + b_ref[...]
        mean = jnp.mean(out, axis=0, keepdims=True)
        var = jnp.mean((out - mean) ** 2, axis=0, keepdims=True)
        o_ref[...] = (out - mean) * lax.rsqrt(var + EPS) * g_ref[...] + be_ref[...]

    return pl.pallas_call(
        body,
        out_shape=jax.ShapeDtypeStruct((N_NODES, HIDDEN_DIM), jnp.float32),
    )(partials, W, b.reshape(1, HIDDEN_DIM), gamma.reshape(1, HIDDEN_DIM),
      beta.reshape(1, HIDDEN_DIM))


def kernel(h, edge_index, W, b, gamma, beta):
    src = edge_index[0].astype(jnp.int32)
    dst = edge_index[1].astype(jnp.int32)
    pad = E_PAD - N_EDGES
    src = jnp.concatenate([src, jnp.zeros((pad,), jnp.int32)])
    dst = jnp.concatenate([dst, jnp.full((pad,), N_NODES, jnp.int32)])
    eidx = jnp.stack([src.reshape(NS, K, C), dst.reshape(NS, K, C)], axis=2)
    h_split = jnp.stack([h[:, :HF], h[:, HF:]])
    zeros = jnp.zeros((ACC_ROWS, HF), jnp.float32)
    partials = _sc_aggregate(h_split, eidx, zeros)
    return _tc_finish(partials, W, b, gamma, beta)
